# Initial kernel scaffold; baseline (speedup 1.0000x reference)
import functools

import jax
import jax.numpy as jnp
from jax import lax
from jax.experimental import pallas as pl
from jax.experimental.pallas import tpu as pltpu
from jax.experimental.pallas import tpu_sc as plsc

NC = 2
NS = 16
L = 16
CH = 128

N = 10000
E = 160000
D = 256
DH = D // NC
NCHUNK = E // CH
ROWS_PER_TILE = N // NS
ZR = 125


def _linear_body(x_ref, w_ref, b_ref, o_ref):
    o_ref[...] = (
        lax.dot_general(x_ref[...], w_ref[...], (((1,), (1,)), ((), ())),
                        preferred_element_type=jnp.float32)
        + b_ref[...]
    )


def _linear(x, W, b):
    bn = 1000
    nb = N // bn
    return pl.pallas_call(
        _linear_body,
        grid=(nb,),
        in_specs=[
            pl.BlockSpec((bn, D), lambda i: (i, 0)),
            pl.BlockSpec((D, D), lambda i: (0, 0)),
            pl.BlockSpec((1, D), lambda i: (0, 0)),
        ],
        out_specs=pl.BlockSpec((bn, D), lambda i: (i, 0)),
        out_shape=jax.ShapeDtypeStruct((N, D), jnp.float32),
    )(x, W, b.reshape(1, D))


def _deg_body(row2d, z1, out, idxb, ones_v, zbuf, deg_sp):
    c = lax.axis_index("c")
    s = lax.axis_index("s")
    wid = c * NS + s

    @pl.when(s == 0)
    def _zero():
        pltpu.sync_copy(z1, zbuf)
        pltpu.sync_copy(zbuf, deg_sp)

    for k in range(CH // L):
        ones_v[pl.ds(k * L, L)] = jnp.ones((L,), jnp.float32)

    nfull = NCHUNK // (NC * NS)
    ntail = NCHUNK - nfull * NC * NS
    pltpu.sync_copy(row2d.at[pl.ds(wid * nfull, nfull)], idxb.at[pl.ds(0, nfull)])

    @pl.when(wid < ntail)
    def _tail():
        pltpu.sync_copy(row2d.at[pl.ds(nfull * NC * NS + wid, 1)],
                        idxb.at[pl.ds(nfull, 1)])

    plsc.subcore_barrier()

    nt = jnp.where(wid < ntail, nfull + 1, nfull)

    def body(j, carry):
        pltpu.sync_copy(ones_v, deg_sp.at[idxb.at[j]], add=True)
        return carry

    lax.fori_loop(0, nt, body, 0)
    plsc.subcore_barrier()

    @pl.when(s == 0)
    def _dump():
        pltpu.sync_copy(deg_sp, zbuf)
        pltpu.sync_copy(zbuf, out.at[c])


def _deg(row2d, z1):
    mesh = plsc.VectorSubcoreMesh(core_axis_name="c", subcore_axis_name="s")
    f = pl.kernel(
        _deg_body,
        out_type=jax.ShapeDtypeStruct((NC, N), jnp.float32),
        mesh=mesh,
        scratch_types=[
            pltpu.VMEM((NCHUNK // (NC * NS) + 1, CH), jnp.int32),
            pltpu.VMEM((CH,), jnp.float32),
            pltpu.VMEM((N,), jnp.float32),
            pltpu.VMEM_SHARED((N,), jnp.float32),
        ],
    )
    return f(row2d, z1)


def _scale_body(degp_ref, x_ref, dis_ref, y2_ref):
    deg = jnp.sum(degp_ref[...], axis=0)
    dis = jnp.where(deg > 0, lax.rsqrt(jnp.maximum(deg, 1.0)), 0.0)
    dis_ref[...] = dis[None, :]
    y2_ref[...] = x_ref[...] * dis[:, None]


def _scale(degp, x_t):
    bn = 1000
    nb = N // bn
    return pl.pallas_call(
        _scale_body,
        grid=(NC, nb),
        in_specs=[
            pl.BlockSpec((NC, bn), lambda c, i: (0, i)),
            pl.BlockSpec((bn, DH), lambda c, i: (i, c)),
        ],
        out_specs=[
            pl.BlockSpec((1, bn), lambda c, i: (c, i)),
            pl.BlockSpec((bn, DH), lambda c, i: (c * nb + i, 0)),
        ],
        out_shape=[
            jax.ShapeDtypeStruct((NC, N), jnp.float32),
            jax.ShapeDtypeStruct((NC * N, DH), jnp.float32),
        ],
    )(degp, x_t)


def _agg_body(y2, colcat, row2d, z2, out, colb, rowb, rows_v, zbuf, acc_sp, sem):
    c = lax.axis_index("c")
    s = lax.axis_index("s")

    pltpu.sync_copy(z2, zbuf)
    for k in range(ROWS_PER_TILE // ZR):
        pltpu.sync_copy(zbuf, acc_sp.at[pl.ds(s * ROWS_PER_TILE + k * ZR, ZR)])

    nfull = NCHUNK // NS
    ntail = NCHUNK - nfull * NS
    pltpu.sync_copy(colcat.at[pl.ds(c * NCHUNK + s * nfull, nfull)],
                    colb.at[pl.ds(0, nfull)])
    pltpu.sync_copy(row2d.at[pl.ds(s * nfull, nfull)], rowb.at[pl.ds(0, nfull)])

    @pl.when(s < ntail)
    def _tail():
        pltpu.sync_copy(colcat.at[pl.ds(c * NCHUNK + nfull * NS + s, 1)],
                        colb.at[pl.ds(nfull, 1)])
        pltpu.sync_copy(row2d.at[pl.ds(nfull * NS + s, 1)],
                        rowb.at[pl.ds(nfull, 1)])

    plsc.subcore_barrier()

    nt = jnp.where(s < ntail, nfull + 1, nfull)

    def body(j, carry):
        pltpu.async_copy(y2.at[colb.at[j]], rows_v, sem).wait()
        pltpu.sync_copy(rows_v, acc_sp.at[rowb.at[j]], add=True)
        return carry

    lax.fori_loop(0, nt, body, 0)
    plsc.subcore_barrier()

    for k in range(ROWS_PER_TILE // ZR):
        r0 = s * ROWS_PER_TILE + k * ZR
        pltpu.sync_copy(acc_sp.at[pl.ds(r0, ZR)], zbuf)
        pltpu.sync_copy(zbuf, out.at[pl.ds(c * N + r0, ZR)])


def _aggregate(y2, colcat, row2d, z2):
    mesh = plsc.VectorSubcoreMesh(core_axis_name="c", subcore_axis_name="s")
    f = pl.kernel(
        _agg_body,
        out_type=jax.ShapeDtypeStruct((NC * N, DH), jnp.float32),
        mesh=mesh,
        scratch_types=[
            pltpu.VMEM((NCHUNK // NS + 1, CH), jnp.int32),
            pltpu.VMEM((NCHUNK // NS + 1, CH), jnp.int32),
            pltpu.VMEM((CH, DH), jnp.float32),
            pltpu.VMEM((ZR, DH), jnp.float32),
            pltpu.VMEM_SHARED((N, DH), jnp.float32),
            pltpu.SemaphoreType.DMA,
        ],
    )
    return f(y2, colcat, row2d, z2)


def _final_body(a0_ref, a1_ref, x_ref, dis_ref, slw_ref, o_ref):
    acc = jnp.concatenate([a0_ref[...], a1_ref[...]], axis=1)
    o_ref[...] = jnp.maximum(
        acc * dis_ref[0, :, None] + x_ref[...] * slw_ref[...], 0.0)


def _final(acc, x_t, dis2, slw):
    bn = 1000
    nb = N // bn
    return pl.pallas_call(
        _final_body,
        grid=(nb,),
        in_specs=[
            pl.BlockSpec((bn, DH), lambda i: (i, 0)),
            pl.BlockSpec((bn, DH), lambda i: (nb + i, 0)),
            pl.BlockSpec((bn, D), lambda i: (i, 0)),
            pl.BlockSpec((1, bn), lambda i: (0, i)),
            pl.BlockSpec((1, D), lambda i: (0, 0)),
        ],
        out_specs=pl.BlockSpec((bn, D), lambda i: (i, 0)),
        out_shape=jax.ShapeDtypeStruct((N, D), jnp.float32),
    )(acc, acc, x_t, dis2, slw.reshape(1, D))


def kernel(x, edge_index, W, b, self_loop_weight):
    row = edge_index[0]
    col = edge_index[1]
    row2d = row.reshape(NCHUNK, CH)
    colcat = jnp.concatenate([col, col + N]).reshape(NC * NCHUNK, CH)
    z1 = jnp.zeros((N,), jnp.float32)
    z2 = jnp.zeros((ZR, DH), jnp.float32)

    x_t = _linear(x, W, b)
    degp = _deg(row2d, z1)
    dis2, y2 = _scale(degp, x_t)
    acc = _aggregate(y2, colcat, row2d, z2)
    return _final(acc, x_t, dis2, self_loop_weight)


# speedup vs baseline: 7.4648x; 7.4648x over previous
import functools

import jax
import jax.numpy as jnp
from jax import lax
from jax.experimental import pallas as pl
from jax.experimental.pallas import tpu as pltpu
from jax.experimental.pallas import tpu_sc as plsc

NC = 2
NS = 16
L = 16
CH = 128

N = 10000
E = 160000
D = 256
DH = D // NC
EP = 163840
NCHP = EP // CH
NPAD = 112
NA = N + NPAD
STRIPE = NA // NS
ZR = 128
CHUNKS = [128, 128, 128, 128, 120]


def _linear_body(x_ref, w_ref, b_ref, o_ref):
    o_ref[...] = (
        lax.dot_general(x_ref[...], w_ref[...], (((1,), (1,)), ((), ())),
                        preferred_element_type=jnp.float32)
        + b_ref[...]
    )


def _linear(x, W, b):
    bn = 1000
    nb = N // bn
    return pl.pallas_call(
        _linear_body,
        grid=(nb,),
        in_specs=[
            pl.BlockSpec((bn, D), lambda i: (i, 0)),
            pl.BlockSpec((D, D), lambda i: (0, 0)),
            pl.BlockSpec((1, D), lambda i: (0, 0)),
        ],
        out_specs=pl.BlockSpec((bn, D), lambda i: (i, 0)),
        out_shape=jax.ShapeDtypeStruct((N, D), jnp.float32),
    )(x, W, b.reshape(1, D))


def _deg_body(row2d, z1, out, idxb, ones_v, zbuf, deg_sp):
    c = lax.axis_index("c")
    s = lax.axis_index("s")
    wid = c * NS + s
    npt = NCHP // (NC * NS)

    @pl.when(s == 0)
    def _zero():
        pltpu.sync_copy(z1, zbuf)
        pltpu.sync_copy(zbuf, deg_sp)

    for k in range(CH // L):
        ones_v[pl.ds(k * L, L)] = jnp.ones((L,), jnp.float32)

    pltpu.sync_copy(row2d.at[pl.ds(wid * npt, npt)], idxb)
    plsc.subcore_barrier()

    def body(j, carry):
        pltpu.sync_copy(ones_v, deg_sp.at[idxb.at[j]], add=True)
        return carry

    lax.fori_loop(0, npt, body, 0)
    plsc.subcore_barrier()

    @pl.when(s == 0)
    def _dump():
        pltpu.sync_copy(deg_sp, zbuf)
        pltpu.sync_copy(zbuf, out.at[c])


def _deg(row2d, z1):
    mesh = plsc.VectorSubcoreMesh(
        core_axis_name="c", subcore_axis_name="s", num_cores=NC, num_subcores=NS
    )
    f = pl.kernel(
        _deg_body,
        out_type=jax.ShapeDtypeStruct((NC, NA), jnp.float32),
        mesh=mesh,
        scratch_types=[
            pltpu.VMEM((NCHP // (NC * NS), CH), jnp.int32),
            pltpu.VMEM((CH,), jnp.float32),
            pltpu.VMEM((NA,), jnp.float32),
            pltpu.VMEM_SHARED((NA,), jnp.float32),
        ],
    )
    return f(row2d, z1)


def _scale_body(deg0_ref, deg1_ref, x_ref, dis_ref, y2_ref):
    deg = deg0_ref[...] + deg1_ref[...]
    dis = jnp.where(deg > 0, lax.rsqrt(jnp.maximum(deg, 1.0)), 0.0)
    dis_ref[...] = dis
    y2_ref[...] = x_ref[...] * dis


def _scale(degp, x_t):
    bn = 1000
    nb = N // bn
    deg0 = degp[0, :N].reshape(N, 1)
    deg1 = degp[1, :N].reshape(N, 1)
    return pl.pallas_call(
        _scale_body,
        grid=(NC, nb),
        in_specs=[
            pl.BlockSpec((bn, 1), lambda c, i: (i, 0)),
            pl.BlockSpec((bn, 1), lambda c, i: (i, 0)),
            pl.BlockSpec((bn, DH), lambda c, i: (i, c)),
        ],
        out_specs=[
            pl.BlockSpec((bn, 1), lambda c, i: (i, 0)),
            pl.BlockSpec((bn, DH), lambda c, i: (c * nb + i, 0)),
        ],
        out_shape=[
            jax.ShapeDtypeStruct((N, 1), jnp.float32),
            jax.ShapeDtypeStruct((NC * N, DH), jnp.float32),
        ],
    )(deg0, deg1, x_t)


def _agg_body(y2, colcat, row2d, z2, out,
              colb, rowb, rows_v, acc_sp, sem):
    c = lax.axis_index("c")
    s = lax.axis_index("s")
    npt = NCHP // NS

    pltpu.sync_copy(z2, rows_v)
    off = 0
    for sz in CHUNKS:
        pltpu.sync_copy(rows_v.at[pl.ds(0, sz)],
                        acc_sp.at[pl.ds(s * STRIPE + off, sz)])
        off += sz

    pltpu.sync_copy(colcat.at[pl.ds(c * NCHP + s * npt, npt)], colb)
    pltpu.sync_copy(row2d.at[pl.ds(s * npt, npt)], rowb)
    plsc.subcore_barrier()

    def body(j, carry):
        pltpu.async_copy(y2.at[colb.at[j]], rows_v, sem).wait()
        pltpu.sync_copy(rows_v, acc_sp.at[rowb.at[j]], add=True)
        return carry

    lax.fori_loop(0, npt, body, 0)
    plsc.subcore_barrier()

    off = 0
    for sz in CHUNKS:
        r0 = s * STRIPE + off
        pltpu.sync_copy(acc_sp.at[pl.ds(r0, sz)], rows_v.at[pl.ds(0, sz)])
        pltpu.sync_copy(rows_v.at[pl.ds(0, sz)], out.at[pl.ds(c * NA + r0, sz)])
        off += sz


def _aggregate(y2, colcat, row2d, z2):
    mesh = plsc.VectorSubcoreMesh(
        core_axis_name="c", subcore_axis_name="s", num_cores=NC, num_subcores=NS
    )
    f = pl.kernel(
        _agg_body,
        out_type=jax.ShapeDtypeStruct((NC * NA, DH), jnp.float32),
        mesh=mesh,
        scratch_types=[
            pltpu.VMEM((NCHP // NS, CH), jnp.int32),
            pltpu.VMEM((NCHP // NS, CH), jnp.int32),
            pltpu.VMEM((CH, DH), jnp.float32),
            pltpu.VMEM_SHARED((NA, DH), jnp.float32),
            pltpu.SemaphoreType.DMA,
        ],
    )
    return f(y2, colcat, row2d, z2)


def _final_body(a0_ref, a1_ref, x_ref, dis_ref, slw_ref, o_ref):
    acc = jnp.concatenate([a0_ref[...], a1_ref[...]], axis=1)
    o_ref[...] = jnp.maximum(
        acc * dis_ref[...] + x_ref[...] * slw_ref[...], 0.0)


def _final(acc0, acc1, x_t, dis2, slw):
    bn = 1000
    nb = N // bn
    return pl.pallas_call(
        _final_body,
        grid=(nb,),
        in_specs=[
            pl.BlockSpec((bn, DH), lambda i: (i, 0)),
            pl.BlockSpec((bn, DH), lambda i: (i, 0)),
            pl.BlockSpec((bn, D), lambda i: (i, 0)),
            pl.BlockSpec((bn, 1), lambda i: (i, 0)),
            pl.BlockSpec((1, D), lambda i: (0, 0)),
        ],
        out_specs=pl.BlockSpec((bn, D), lambda i: (i, 0)),
        out_shape=jax.ShapeDtypeStruct((N, D), jnp.float32),
    )(acc0, acc1, x_t, dis2, slw.reshape(1, D))


def kernel(x, edge_index, W, b, self_loop_weight):
    row = edge_index[0]
    col = edge_index[1]
    npad_e = EP - E
    pad_rows = N + (jnp.arange(npad_e, dtype=jnp.int32) % NPAD)
    rowp = jnp.concatenate([row, pad_rows]).reshape(NCHP, CH)
    colp = jnp.concatenate([col, jnp.zeros(npad_e, jnp.int32)])
    colcat = jnp.concatenate([colp, colp + N]).reshape(NC * NCHP, CH)
    z1 = jnp.zeros((NA,), jnp.float32)
    z2 = jnp.zeros((ZR, DH), jnp.float32)

    x_t = _linear(x, W, b)
    degp = _deg(rowp, z1)
    dis2, y2 = _scale(degp, x_t)
    accfull = _aggregate(y2, colcat, rowp, z2)
    acc1 = lax.slice(accfull, (NA, 0), (NA + N, DH))
    return _final(accfull, acc1, x_t, dis2, self_loop_weight)


# speedup vs baseline: 8.8315x; 1.1831x over previous
import functools

import jax
import jax.numpy as jnp
from jax import lax
from jax.experimental import pallas as pl
from jax.experimental.pallas import tpu as pltpu
from jax.experimental.pallas import tpu_sc as plsc

NC = 2
NS = 16
L = 16
CH = 128

N = 10000
E = 160000
D = 256
DH = D // NC
EP = 163840
NCHP = EP // CH
NPAD = 112
NA = N + NPAD
STRIPE = NA // NS
ZR = 128
CHUNKS = [128, 128, 128, 128, 120]


def _linear_body(x_ref, w_ref, b_ref, o_ref):
    o_ref[...] = (
        lax.dot_general(x_ref[...], w_ref[...], (((1,), (1,)), ((), ())),
                        preferred_element_type=jnp.float32)
        + b_ref[...]
    )


def _linear(x, W, b):
    bn = 1000
    nb = N // bn
    return pl.pallas_call(
        _linear_body,
        grid=(nb,),
        in_specs=[
            pl.BlockSpec((bn, D), lambda i: (i, 0)),
            pl.BlockSpec((D, D), lambda i: (0, 0)),
            pl.BlockSpec((1, D), lambda i: (0, 0)),
        ],
        out_specs=pl.BlockSpec((bn, D), lambda i: (i, 0)),
        out_shape=jax.ShapeDtypeStruct((N, D), jnp.float32),
    )(x, W, b.reshape(1, D))


def _deg_body(row2d, z1, out, idxb, ones_v, zbuf, deg_sp):
    c = lax.axis_index("c")
    s = lax.axis_index("s")
    wid = c * NS + s
    npt = NCHP // (NC * NS)

    @pl.when(s == 0)
    def _zero():
        pltpu.sync_copy(z1, zbuf)
        pltpu.sync_copy(zbuf, deg_sp)

    for k in range(CH // L):
        ones_v[pl.ds(k * L, L)] = jnp.ones((L,), jnp.float32)

    pltpu.sync_copy(row2d.at[pl.ds(wid * npt, npt)], idxb)
    plsc.subcore_barrier()

    def body(j, carry):
        pltpu.sync_copy(ones_v, deg_sp.at[idxb.at[j]], add=True)
        return carry

    lax.fori_loop(0, npt, body, 0)
    plsc.subcore_barrier()

    @pl.when(s == 0)
    def _dump():
        pltpu.sync_copy(deg_sp, zbuf)
        pltpu.sync_copy(zbuf, out.at[c])


def _deg(row2d, z1):
    mesh = plsc.VectorSubcoreMesh(
        core_axis_name="c", subcore_axis_name="s", num_cores=NC, num_subcores=NS
    )
    f = pl.kernel(
        _deg_body,
        out_type=jax.ShapeDtypeStruct((NC, NA), jnp.float32),
        mesh=mesh,
        scratch_types=[
            pltpu.VMEM((NCHP // (NC * NS), CH), jnp.int32),
            pltpu.VMEM((CH,), jnp.float32),
            pltpu.VMEM((NA,), jnp.float32),
            pltpu.VMEM_SHARED((NA,), jnp.float32),
        ],
    )
    return f(row2d, z1)


def _scale_body(deg0_ref, deg1_ref, x_ref, dis_ref, y2_ref):
    deg = deg0_ref[...] + deg1_ref[...]
    dis = jnp.where(deg > 0, lax.rsqrt(jnp.maximum(deg, 1.0)), 0.0)
    dis_ref[...] = dis
    y2_ref[...] = x_ref[...] * dis


def _scale(degp, x_t):
    bn = 1000
    nb = N // bn
    deg0 = degp[0, :N].reshape(N, 1)
    deg1 = degp[1, :N].reshape(N, 1)
    return pl.pallas_call(
        _scale_body,
        grid=(NC, nb),
        in_specs=[
            pl.BlockSpec((bn, 1), lambda c, i: (i, 0)),
            pl.BlockSpec((bn, 1), lambda c, i: (i, 0)),
            pl.BlockSpec((bn, DH), lambda c, i: (i, c)),
        ],
        out_specs=[
            pl.BlockSpec((bn, 1), lambda c, i: (i, 0)),
            pl.BlockSpec((bn, DH), lambda c, i: (c * nb + i, 0)),
        ],
        out_shape=[
            jax.ShapeDtypeStruct((N, 1), jnp.float32),
            jax.ShapeDtypeStruct((NC * N, DH), jnp.float32),
        ],
    )(deg0, deg1, x_t)


def _agg_body(y2, colcat, row2d, z2, out,
              colb, rowb, rows_a, rows_b, acc_sp, sem_a, sem_b):
    c = lax.axis_index("c")
    s = lax.axis_index("s")
    npt = NCHP // NS

    pltpu.sync_copy(z2, rows_a)
    off = 0
    for sz in CHUNKS:
        pltpu.sync_copy(rows_a.at[pl.ds(0, sz)],
                        acc_sp.at[pl.ds(s * STRIPE + off, sz)])
        off += sz

    plsc.subcore_barrier()

    nph = npt // 2
    nhalf = nph // 2
    for h in range(2):
        pltpu.sync_copy(colcat.at[pl.ds(c * NCHP + s * npt + h * nph, nph)],
                        colb)
        pltpu.sync_copy(row2d.at[pl.ds(s * npt + h * nph, nph)], rowb)
        pltpu.async_copy(y2.at[colb.at[0]], rows_a, sem_a)

        def body(i, carry):
            ja = 2 * i
            jb = 2 * i + 1
            pltpu.async_copy(y2.at[colb.at[jb]], rows_b, sem_b)
            pltpu.make_async_copy(y2.at[colb.at[ja]], rows_a, sem_a).wait()
            pltpu.sync_copy(rows_a, acc_sp.at[rowb.at[ja]], add=True)

            @pl.when(i < nhalf - 1)
            def _next():
                pltpu.async_copy(y2.at[colb.at[ja + 2]], rows_a, sem_a)

            pltpu.make_async_copy(y2.at[colb.at[jb]], rows_b, sem_b).wait()
            pltpu.sync_copy(rows_b, acc_sp.at[rowb.at[jb]], add=True)
            return carry

        lax.fori_loop(0, nhalf, body, 0)
    plsc.subcore_barrier()

    off = 0
    for sz in CHUNKS:
        r0 = s * STRIPE + off
        pltpu.sync_copy(acc_sp.at[pl.ds(r0, sz)], rows_a.at[pl.ds(0, sz)])
        pltpu.sync_copy(rows_a.at[pl.ds(0, sz)], out.at[pl.ds(c * NA + r0, sz)])
        off += sz


def _aggregate(y2, colcat, row2d, z2):
    mesh = plsc.VectorSubcoreMesh(
        core_axis_name="c", subcore_axis_name="s", num_cores=NC, num_subcores=NS
    )
    f = pl.kernel(
        _agg_body,
        out_type=jax.ShapeDtypeStruct((NC * NA, DH), jnp.float32),
        mesh=mesh,
        scratch_types=[
            pltpu.VMEM((NCHP // NS // 2, CH), jnp.int32),
            pltpu.VMEM((NCHP // NS // 2, CH), jnp.int32),
            pltpu.VMEM((CH, DH), jnp.float32),
            pltpu.VMEM((CH, DH), jnp.float32),
            pltpu.VMEM_SHARED((NA, DH), jnp.float32),
            pltpu.SemaphoreType.DMA,
            pltpu.SemaphoreType.DMA,
        ],
    )
    return f(y2, colcat, row2d, z2)


def _final_body(a0_ref, a1_ref, x_ref, dis_ref, slw_ref, o_ref):
    acc = jnp.concatenate([a0_ref[...], a1_ref[...]], axis=1)
    o_ref[...] = jnp.maximum(
        acc * dis_ref[...] + x_ref[...] * slw_ref[...], 0.0)


def _final(acc0, acc1, x_t, dis2, slw):
    bn = 1000
    nb = N // bn
    return pl.pallas_call(
        _final_body,
        grid=(nb,),
        in_specs=[
            pl.BlockSpec((bn, DH), lambda i: (i, 0)),
            pl.BlockSpec((bn, DH), lambda i: (i, 0)),
            pl.BlockSpec((bn, D), lambda i: (i, 0)),
            pl.BlockSpec((bn, 1), lambda i: (i, 0)),
            pl.BlockSpec((1, D), lambda i: (0, 0)),
        ],
        out_specs=pl.BlockSpec((bn, D), lambda i: (i, 0)),
        out_shape=jax.ShapeDtypeStruct((N, D), jnp.float32),
    )(acc0, acc1, x_t, dis2, slw.reshape(1, D))


def kernel(x, edge_index, W, b, self_loop_weight):
    row = edge_index[0]
    col = edge_index[1]
    npad_e = EP - E
    pad_rows = N + (jnp.arange(npad_e, dtype=jnp.int32) % NPAD)
    rowp = jnp.concatenate([row, pad_rows]).reshape(NCHP, CH)
    colp = jnp.concatenate([col, jnp.zeros(npad_e, jnp.int32)])
    colcat = jnp.concatenate([colp, colp + N]).reshape(NC * NCHP, CH)
    z1 = jnp.zeros((NA,), jnp.float32)
    z2 = jnp.zeros((ZR, DH), jnp.float32)

    x_t = _linear(x, W, b)
    degp = _deg(rowp, z1)
    dis2, y2 = _scale(degp, x_t)
    accfull = _aggregate(y2, colcat, rowp, z2)
    acc1 = lax.slice(accfull, (NA, 0), (NA + N, DH))
    return _final(accfull, acc1, x_t, dis2, self_loop_weight)
